# baseline (speedup 1.0000x reference)
import functools

import jax
import jax.numpy as jnp
from jax import lax
from jax.experimental import pallas as pl
from jax.experimental.pallas import tpu as pltpu
from jax.experimental.pallas import tpu_sc as plsc

B = 16384
H = 128
POS = 32
L = 16
NC = 2
NS = 16
NW = NC * NS
CH = 64
CPW = B // (NW * CH)
NV = H // L
EPS = 1e-12

_GDN = lax.GatherDimensionNumbers(
    offset_dims=(), collapsed_slice_dims=(0,), start_index_map=(0,))


def _perm(v, idx):
    return lax.gather(v, idx[:, None], _GDN, slice_sizes=(1,),
                      mode=lax.GatherScatterMode.PROMISE_IN_BOUNDS)


def _merge(a, b, sh):
    l = lax.iota(jnp.int32, L)
    m = (l & sh) != 0
    pa = _perm(a, l ^ sh)
    pb = _perm(b, l ^ sh)
    return jnp.where(m, pb, a) + jnp.where(m, b, pa)


def _bcast(v, k):
    return _perm(v, jnp.full((L,), k, jnp.int32))


_BITREV = (0, 8, 4, 12, 2, 10, 6, 14, 1, 9, 5, 13, 3, 11, 7, 15)


def _rsqrt_vec(v):
    i = lax.bitcast_convert_type(v, jnp.int32)
    i = jnp.int32(0x5F3759DF) - (i >> 1)
    y = lax.bitcast_convert_type(i, jnp.float32)
    for _ in range(2):
        y = y * (1.5 - 0.5 * v * y * y)
    return y


def _sc_body(xT, syn, pos, sen, lem, gam, bet, out,
             idx_v, bufA, bufB, bufC, bufD, outA, outB, pos_v, g_v, b_v,
             semA, semB, semC, semD, semO):
    wid = lax.axis_index("s") * NC + lax.axis_index("c")
    cbase = wid * CPW
    for t in range(4):
        pltpu.sync_copy(xT.at[t, pl.ds(cbase, CPW)], idx_v.at[t])
    pltpu.sync_copy(pos, pos_v)
    pltpu.sync_copy(gam, g_v)
    pltpu.sync_copy(bet, b_v)

    bufs = (bufA, bufB, bufC, bufD)
    outs = (outA, outB)
    sems = (semA, semB, semC, semD)

    def write_gather(c):
        return pltpu.async_copy(syn.at[idx_v.at[0, c]], bufs[c % 4],
                                sems[c % 4])

    def add_gathers(c):
        return [pltpu.async_copy(tab.at[idx_v.at[t, c]], bufs[c % 4],
                                 sems[c % 4], add=True)
                for t, tab in ((2, sen), (3, lem))]

    pend_wg = {}
    pend_add = {}
    ostores = {}
    for c in range(min(2, CPW)):
        pend_wg[c] = write_gather(c)
    for c in range(min(2, CPW)):
        pend_wg.pop(c).wait()
        pend_add[c] = add_gathers(c)

    for c in range(CPW):
        buf, ob = bufs[c % 4], outs[c % 2]
        for cp in pend_add.pop(c):
            cp.wait()
        if c - 2 in ostores:
            ostores.pop(c - 2).wait()
        if c + 2 < CPW:
            pend_wg[c + 2] = write_gather(c + 2)

        def group(g, gb):
            base = g * L
            pidx = idx_v[1, c, pl.ds(base, L)]
            stack = []
            for j in range(L):
                k = _BITREV[j]
                r = base + k
                pi = pidx[k]
                s = None
                q = None
                for jj in range(NV):
                    v = buf[r, pl.ds(jj * L, L)] + pos_v[pi, pl.ds(jj * L, L)]
                    buf[r, pl.ds(jj * L, L)] = v
                    s = v if s is None else s + v
                    p = v * v
                    q = p if q is None else q + p
                node = (0, s, q)
                while stack and stack[-1][0] == node[0]:
                    lv, s2, q2 = stack.pop()
                    sh = (8, 4, 2, 1)[lv]
                    node = (lv + 1, _merge(s2, node[1], sh),
                            _merge(q2, node[2], sh))
                stack.append(node)
            _, sT, qT = stack[0]
            mean = sT * (1.0 / H)
            var = qT * (1.0 / H) - mean * mean
            rstd = _rsqrt_vec(var + EPS)
            for k in range(L):
                mk = _bcast(mean, k)
                rk = _bcast(rstd, k)
                r = base + k
                for j in range(NV):
                    ob[r, pl.ds(j * L, L)] = \
                        (buf[r, pl.ds(j * L, L)] - mk) * rk * gb[j] \
                        + gb[NV + j]
            return gb

        gb0 = tuple(g_v[pl.ds(j * L, L)] for j in range(NV)) + \
            tuple(b_v[pl.ds(j * L, L)] for j in range(NV))
        lax.fori_loop(0, CH // L, group, gb0)

        if c + 2 < CPW:
            pend_wg.pop(c + 2).wait()
            pend_add[c + 2] = add_gathers(c + 2)
        ostores[c] = pltpu.async_copy(
            ob, out.at[pl.ds((cbase + c) * CH, CH)], semO)
    for cp in ostores.values():
        cp.wait()


_mesh = plsc.VectorSubcoreMesh(core_axis_name="c", subcore_axis_name="s")

_embed_ln = functools.partial(
    pl.kernel,
    out_type=jax.ShapeDtypeStruct((B, H), jnp.float32),
    mesh=_mesh,
    scratch_types=[
        pltpu.VMEM((4, CPW, CH), jnp.int32),
        pltpu.VMEM((CH, H), jnp.float32),
        pltpu.VMEM((CH, H), jnp.float32),
        pltpu.VMEM((CH, H), jnp.float32),
        pltpu.VMEM((CH, H), jnp.float32),
        pltpu.VMEM((CH, H), jnp.float32),
        pltpu.VMEM((CH, H), jnp.float32),
        pltpu.VMEM((POS, H), jnp.float32),
        pltpu.VMEM((H,), jnp.float32),
        pltpu.VMEM((H,), jnp.float32),
        pltpu.SemaphoreType.DMA,
        pltpu.SemaphoreType.DMA,
        pltpu.SemaphoreType.DMA,
        pltpu.SemaphoreType.DMA,
        pltpu.SemaphoreType.DMA,
    ],
)(_sc_body)


@jax.jit
def kernel(x, synset_table, pos_table, sense_table, lemma_table,
           ln_gamma, ln_beta):
    xT = jnp.asarray(x, jnp.int32).T.reshape(4, B // CH, CH)
    return _embed_ln(xT, synset_table, pos_table, sense_table, lemma_table,
                     ln_gamma, ln_beta)


# speedup vs baseline: 1.0395x; 1.0395x over previous
import functools

import jax
import jax.numpy as jnp
from jax import lax
from jax.experimental import pallas as pl
from jax.experimental.pallas import tpu as pltpu
from jax.experimental.pallas import tpu_sc as plsc

B = 16384
H = 128
POS = 32
L = 16
NC = 2
NS = 16
NW = NC * NS
CH = 128
CPW = B // (NW * CH)
NV = H // L
EPS = 1e-12

_GDN = lax.GatherDimensionNumbers(
    offset_dims=(), collapsed_slice_dims=(0,), start_index_map=(0,))


def _perm(v, idx):
    return lax.gather(v, idx[:, None], _GDN, slice_sizes=(1,),
                      mode=lax.GatherScatterMode.PROMISE_IN_BOUNDS)


def _merge(a, b, sh):
    l = lax.iota(jnp.int32, L)
    m = (l & sh) != 0
    pa = _perm(a, l ^ sh)
    pb = _perm(b, l ^ sh)
    return jnp.where(m, pb, a) + jnp.where(m, b, pa)


def _bcast(v, k):
    return _perm(v, jnp.full((L,), k, jnp.int32))


_BITREV = (0, 8, 4, 12, 2, 10, 6, 14, 1, 9, 5, 13, 3, 11, 7, 15)


def _rsqrt_vec(v):
    i = lax.bitcast_convert_type(v, jnp.int32)
    i = jnp.int32(0x5F3759DF) - (i >> 1)
    y = lax.bitcast_convert_type(i, jnp.float32)
    for _ in range(2):
        y = y * (1.5 - 0.5 * v * y * y)
    return y


def _sc_body(xT, syn, pos, sen, lem, gam, bet, out,
             idx_v, bufA, bufB, bufC, bufD, outA, outB, pos_v, g_v, b_v,
             semA, semB, semC, semD, semO):
    wid = lax.axis_index("s") * NC + lax.axis_index("c")
    cbase = wid * CPW
    for t in range(4):
        pltpu.sync_copy(xT.at[t, pl.ds(cbase, CPW)], idx_v.at[t])
    pltpu.sync_copy(pos, pos_v)
    pltpu.sync_copy(gam, g_v)
    pltpu.sync_copy(bet, b_v)

    bufs = (bufA, bufB, bufC, bufD)
    outs = (outA, outB)
    sems = (semA, semB, semC, semD)

    def write_gather(c):
        return pltpu.async_copy(syn.at[idx_v.at[0, c]], bufs[c], sems[c])

    def add_gathers(c):
        return [pltpu.async_copy(tab.at[idx_v.at[t, c]], bufs[c], sems[c],
                                 add=True)
                for t, tab in ((2, sen), (3, lem))]

    pend_wg = {}
    pend_add = {}
    ostores = {}
    for c in range(min(2, CPW)):
        pend_wg[c] = write_gather(c)
    for c in range(min(2, CPW)):
        pend_wg.pop(c).wait()
        pend_add[c] = add_gathers(c)

    for c in range(CPW):
        buf, ob = bufs[c], outs[c % 2]
        for cp in pend_add.pop(c):
            cp.wait()
        if c - 2 in ostores:
            ostores.pop(c - 2).wait()
        if c + 2 < CPW:
            pend_wg[c + 2] = write_gather(c + 2)

        def group(g, gb):
            base = g * L
            pidx = idx_v[1, c, pl.ds(base, L)]
            stack = []
            for j in range(L):
                k = _BITREV[j]
                r = base + k
                pi = pidx[k]
                s = None
                q = None
                for jj in range(NV):
                    v = buf[r, pl.ds(jj * L, L)] + pos_v[pi, pl.ds(jj * L, L)]
                    buf[r, pl.ds(jj * L, L)] = v
                    s = v if s is None else s + v
                    p = v * v
                    q = p if q is None else q + p
                node = (0, s, q)
                while stack and stack[-1][0] == node[0]:
                    lv, s2, q2 = stack.pop()
                    sh = (8, 4, 2, 1)[lv]
                    node = (lv + 1, _merge(s2, node[1], sh),
                            _merge(q2, node[2], sh))
                stack.append(node)
            _, sT, qT = stack[0]
            mean = sT * (1.0 / H)
            var = qT * (1.0 / H) - mean * mean
            rstd = _rsqrt_vec(var + EPS)
            for k in range(L):
                mk = _bcast(mean, k)
                rk = _bcast(rstd, k)
                r = base + k
                for j in range(NV):
                    ob[r, pl.ds(j * L, L)] = \
                        (buf[r, pl.ds(j * L, L)] - mk) * rk * gb[j] \
                        + gb[NV + j]
            return gb

        gb0 = tuple(g_v[pl.ds(j * L, L)] for j in range(NV)) + \
            tuple(b_v[pl.ds(j * L, L)] for j in range(NV))
        lax.fori_loop(0, CH // L, group, gb0)

        if c + 2 < CPW:
            pend_wg.pop(c + 2).wait()
            pend_add[c + 2] = add_gathers(c + 2)
        ostores[c] = pltpu.async_copy(
            ob, out.at[pl.ds((cbase + c) * CH, CH)], semO)
    for cp in ostores.values():
        cp.wait()


_mesh = plsc.VectorSubcoreMesh(core_axis_name="c", subcore_axis_name="s")

_embed_ln = functools.partial(
    pl.kernel,
    out_type=jax.ShapeDtypeStruct((B, H), jnp.float32),
    mesh=_mesh,
    scratch_types=[
        pltpu.VMEM((4, CPW, CH), jnp.int32),
        pltpu.VMEM((CH, H), jnp.float32),
        pltpu.VMEM((CH, H), jnp.float32),
        pltpu.VMEM((CH, H), jnp.float32),
        pltpu.VMEM((CH, H), jnp.float32),
        pltpu.VMEM((CH, H), jnp.float32),
        pltpu.VMEM((CH, H), jnp.float32),
        pltpu.VMEM((POS, H), jnp.float32),
        pltpu.VMEM((H,), jnp.float32),
        pltpu.VMEM((H,), jnp.float32),
        pltpu.SemaphoreType.DMA,
        pltpu.SemaphoreType.DMA,
        pltpu.SemaphoreType.DMA,
        pltpu.SemaphoreType.DMA,
        pltpu.SemaphoreType.DMA,
    ],
)(_sc_body)


@jax.jit
def kernel(x, synset_table, pos_table, sense_table, lemma_table,
           ln_gamma, ln_beta):
    xT = jnp.asarray(x, jnp.int32).T.reshape(4, B // CH, CH)
    return _embed_ln(xT, synset_table, pos_table, sense_table, lemma_table,
                     ln_gamma, ln_beta)
